# Initial kernel scaffold; baseline (speedup 1.0000x reference)
import jax
import jax.numpy as jnp
from jax.experimental import pallas as pl


def kernel(mf_values, rule_indices):
    raise NotImplementedError("write your pallas kernel here")


# speedup vs baseline: 11800.7153x; 11800.7153x over previous
import jax
import jax.numpy as jnp
from jax.experimental import pallas as pl


def _rule_kernel(mf_ref, idx_ref, firing_ref, norm_ref):
    idx = idx_ref[...]
    num_m = mf_ref.shape[0]
    acc = None
    for m in range(num_m):
        logm = jnp.log(mf_ref[m] + 1e-9)
        mask = (idx == m).astype(jnp.float32)
        part = jax.lax.dot_general(
            logm, mask, (((1,), (1,)), ((), ())),
            preferred_element_type=jnp.float32,
            precision=jax.lax.Precision.HIGHEST)
        acc = part if acc is None else acc + part
    firing = jnp.exp(acc)
    s = jnp.sum(firing, axis=1, keepdims=True) + 1e-6
    firing_ref[...] = firing
    norm_ref[...] = firing / s


def kernel(mf_values, rule_indices):
    b, f, m = mf_values.shape
    r = rule_indices.shape[0]
    mf_t = jnp.transpose(mf_values, (2, 0, 1))
    idx = rule_indices.astype(jnp.int32)
    firing, norm = pl.pallas_call(
        _rule_kernel,
        out_shape=(jax.ShapeDtypeStruct((b, r), jnp.float32),
                   jax.ShapeDtypeStruct((b, r), jnp.float32)),
    )(mf_t, idx)
    return firing, norm


# speedup vs baseline: 22054.1934x; 1.8689x over previous
import jax
import jax.numpy as jnp
from jax.experimental import pallas as pl


def _rule_kernel(mf_ref, idxt_ref, firing_ref, norm_ref):
    f, r = idxt_ref.shape
    k = mf_ref.shape[1]
    m = k // f
    idxt = idxt_ref[...]
    idx_exp = jnp.broadcast_to(idxt[:, None, :], (f, m, r)).reshape(k, r)
    m_of_k = jax.lax.broadcasted_iota(jnp.int32, (k, 1), 0) % m
    w = (idx_exp == m_of_k).astype(jnp.float32)
    logs = jnp.log(mf_ref[...] + 1e-9)
    log_firing = jax.lax.dot_general(
        logs, w, (((1,), (0,)), ((), ())),
        preferred_element_type=jnp.float32,
        precision=jax.lax.Precision.HIGHEST)
    firing = jnp.exp(log_firing)
    s = jnp.sum(firing, axis=1, keepdims=True) + 1e-6
    firing_ref[...] = firing
    norm_ref[...] = firing / s


def kernel(mf_values, rule_indices):
    b, f, m = mf_values.shape
    r = rule_indices.shape[0]
    mf_flat = jnp.reshape(mf_values, (b, f * m))
    idxt = rule_indices.astype(jnp.int32).T
    firing, norm = pl.pallas_call(
        _rule_kernel,
        out_shape=(jax.ShapeDtypeStruct((b, r), jnp.float32),
                   jax.ShapeDtypeStruct((b, r), jnp.float32)),
    )(mf_flat, idxt)
    return firing, norm
